# baseline (speedup 1.0000x reference)
import functools

import jax
import jax.numpy as jnp
from jax import lax
from jax.experimental import pallas as pl
from jax.experimental.pallas import tpu as pltpu
from jax.experimental.pallas import tpu_sc as plsc

NC = 2
NS = 16
NW = NC * NS


def _sc_gather_chunk(table, tokens2d, c, n_chunks):
    V, D = table.shape
    Bt, L = tokens2d.shape
    seg = L // n_chunks
    Bc = Bt * seg
    tpw = Bc // NW
    CH = min(tpw, 64)
    n_ch = tpw // CH
    mesh = plsc.VectorSubcoreMesh(core_axis_name="c", subcore_axis_name="s")

    @functools.partial(
        pl.kernel,
        mesh=mesh,
        out_type=jax.ShapeDtypeStruct((Bc, D), jnp.float32),
        scratch_types=[
            pltpu.VMEM((tpw,), jnp.int32),
            pltpu.VMEM((CH, D), jnp.float32),
            pltpu.VMEM((CH, D), jnp.float32),
            pltpu.SemaphoreType.DMA,
            pltpu.SemaphoreType.DMA,
            pltpu.SemaphoreType.DMA,
            pltpu.SemaphoreType.DMA,
        ],
    )
    def gather_kernel(table_hbm, tok_hbm, out_hbm, idx_v,
                      rows0, rows1, gsem0, gsem1, ssem0, ssem1):
        wid = lax.axis_index("s") * NC + lax.axis_index("c")
        base = wid * tpw
        bi = base // seg
        col0 = c * seg + base % seg
        pltpu.sync_copy(tok_hbm.at[bi, pl.ds(col0, tpw)], idx_v)

        bufs = [(rows0, gsem0, ssem0), (rows1, gsem1, ssem1)]

        def start_gather(k):
            buf, gsem, _ = bufs[k % 2]
            return pltpu.async_copy(
                table_hbm.at[idx_v.at[pl.ds(k * CH, CH)]], buf, gsem)

        gathers = [None, None]
        stores = [None, None]
        gathers[0] = start_gather(0)
        if n_ch > 1:
            gathers[1] = start_gather(1)
        for k in range(n_ch):
            p = k % 2
            buf, _, ssem = bufs[p]
            gathers[p].wait()
            stores[p] = pltpu.async_copy(
                buf, out_hbm.at[pl.ds(base + k * CH, CH)], ssem)
            if k + 2 < n_ch:
                stores[p].wait()
                gathers[p] = start_gather(k + 2)
        for h in stores:
            if h is not None:
                h.wait()

    return gather_kernel(table, tokens2d)


def _tc_project_chunk(emb, Wb, b2, position, B, c, n_chunks, dest):
    Bc, D = emb.shape
    M = Wb.shape[0]
    L = position.shape[0]
    seg = L // n_chunks
    nb = Bc // seg

    def mm_kernel(*refs):
        emb_ref, w_ref, b_ref, pos_ref, out_ref = refs[-5:]
        acc = lax.dot_general(
            emb_ref[...].astype(jnp.bfloat16), w_ref[...],
            dimension_numbers=(((1,), (1,)), ((), ())),
            preferred_element_type=jnp.float32,
        )
        out_ref[...] = acc + b_ref[...] + pos_ref[...]

    MB = 256
    nn = M // MB
    data_specs = [
        pl.BlockSpec((seg, D), lambda k, n: (k, 0)),
        pl.BlockSpec((MB, D), lambda k, n: (n, 0)),
        pl.BlockSpec((1, MB), lambda k, n: (0, n)),
        pl.BlockSpec((seg, MB), lambda k, n: (c, n)),
    ]
    if dest is None:
        in_specs, aliases, args = data_specs, {}, (emb, Wb, b2, position)
    else:
        in_specs = [pl.BlockSpec(memory_space=pl.ANY)] + data_specs
        aliases = {0: 0}
        args = (dest, emb, Wb, b2, position)

    return pl.pallas_call(
        mm_kernel,
        grid=(nb, nn),
        in_specs=in_specs,
        out_specs=pl.BlockSpec((seg, MB), lambda k, n: (k * n_chunks + c, n)),
        out_shape=jax.ShapeDtypeStruct((B, M), jnp.float32),
        input_output_aliases=aliases,
    )(*args)


def kernel(tokens, table, W, b, position):
    batch, seq = tokens.shape
    M = W.shape[0]
    B = batch * seq
    tokens2d = tokens.astype(jnp.int32)
    Wb = W.astype(jnp.bfloat16)
    b2 = b.reshape(1, M)

    n_chunks = 1
    embs = [_sc_gather_chunk(table, tokens2d, c, n_chunks)
            for c in range(n_chunks)]
    out = None
    for c in range(n_chunks):
        out = _tc_project_chunk(embs[c], Wb, b2, position, B, c, n_chunks, out)
    return out.reshape(batch, seq, M)


# speedup vs baseline: 1.2570x; 1.2570x over previous
import functools

import jax
import jax.numpy as jnp
from jax import lax
from jax.experimental import pallas as pl
from jax.experimental.pallas import tpu as pltpu
from jax.experimental.pallas import tpu_sc as plsc

NC = 2
NS = 16
NW = NC * NS


def _sc_gather_chunk(table, tokens2d, c, n_chunks):
    V, D = table.shape
    Bt, L = tokens2d.shape
    seg = L // n_chunks
    Bc = Bt * seg
    tpw = Bc // NW
    CH = min(tpw, 64)
    n_ch = tpw // CH
    mesh = plsc.VectorSubcoreMesh(core_axis_name="c", subcore_axis_name="s")

    @functools.partial(
        pl.kernel,
        mesh=mesh,
        out_type=jax.ShapeDtypeStruct((Bc, D), jnp.float32),
        scratch_types=[
            pltpu.VMEM((tpw,), jnp.int32),
            pltpu.VMEM((CH, D), jnp.float32),
            pltpu.VMEM((CH, D), jnp.float32),
            pltpu.SemaphoreType.DMA,
            pltpu.SemaphoreType.DMA,
            pltpu.SemaphoreType.DMA,
            pltpu.SemaphoreType.DMA,
        ],
    )
    def gather_kernel(table_hbm, tok_hbm, out_hbm, idx_v,
                      rows0, rows1, gsem0, gsem1, ssem0, ssem1):
        wid = lax.axis_index("s") * NC + lax.axis_index("c")
        base = wid * tpw
        bi = base // seg
        col0 = c * seg + base % seg
        pltpu.sync_copy(tok_hbm.at[bi, pl.ds(col0, tpw)], idx_v)

        bufs = [(rows0, gsem0, ssem0), (rows1, gsem1, ssem1)]

        def start_gather(k):
            buf, gsem, _ = bufs[k % 2]
            return pltpu.async_copy(
                table_hbm.at[idx_v.at[pl.ds(k * CH, CH)]], buf, gsem)

        gathers = [None, None]
        stores = [None, None]
        gathers[0] = start_gather(0)
        if n_ch > 1:
            gathers[1] = start_gather(1)
        for k in range(n_ch):
            p = k % 2
            buf, _, ssem = bufs[p]
            gathers[p].wait()
            stores[p] = pltpu.async_copy(
                buf, out_hbm.at[pl.ds(base + k * CH, CH)], ssem)
            if k + 2 < n_ch:
                stores[p].wait()
                gathers[p] = start_gather(k + 2)
        for h in stores:
            if h is not None:
                h.wait()

    return gather_kernel(table, tokens2d)


def _tc_project_chunk(emb, Wb, b2, position, B, c, n_chunks, dest):
    Bc, D = emb.shape
    M = Wb.shape[0]
    L = position.shape[0]
    seg = L // n_chunks
    nb = Bc // seg

    def mm_kernel(*refs):
        emb_ref, w_ref, b_ref, pos_ref, out_ref = refs[-5:]
        acc = lax.dot_general(
            emb_ref[...].astype(jnp.bfloat16), w_ref[...],
            dimension_numbers=(((1,), (1,)), ((), ())),
            preferred_element_type=jnp.float32,
        )
        out_ref[...] = acc + b_ref[...] + pos_ref[...]

    data_specs = [
        pl.BlockSpec((seg, D), lambda k: (k, 0)),
        pl.BlockSpec((M, D), lambda k: (0, 0)),
        pl.BlockSpec((1, M), lambda k: (0, 0)),
        pl.BlockSpec((seg, M), lambda k: (c, 0)),
    ]
    if dest is None:
        in_specs, aliases, args = data_specs, {}, (emb, Wb, b2, position)
    else:
        in_specs = [pl.BlockSpec(memory_space=pl.ANY)] + data_specs
        aliases = {0: 0}
        args = (dest, emb, Wb, b2, position)

    return pl.pallas_call(
        mm_kernel,
        grid=(nb,),
        in_specs=in_specs,
        out_specs=pl.BlockSpec((seg, M), lambda k: (k * n_chunks + c, 0)),
        out_shape=jax.ShapeDtypeStruct((B, M), jnp.float32),
        input_output_aliases=aliases,
    )(*args)


def kernel(tokens, table, W, b, position):
    batch, seq = tokens.shape
    M = W.shape[0]
    B = batch * seq
    tokens2d = tokens.astype(jnp.int32)
    Wb = W.astype(jnp.bfloat16)
    b2 = b.reshape(1, M)

    n_chunks = 1
    embs = [_sc_gather_chunk(table, tokens2d, c, n_chunks)
            for c in range(n_chunks)]
    out = None
    for c in range(n_chunks):
        out = _tc_project_chunk(embs[c], Wb, b2, position, B, c, n_chunks, out)
    return out.reshape(batch, seq, M)


# speedup vs baseline: 1.2670x; 1.0080x over previous
import functools

import jax
import jax.numpy as jnp
from jax import lax
from jax.experimental import pallas as pl
from jax.experimental.pallas import tpu as pltpu
from jax.experimental.pallas import tpu_sc as plsc

NC = 2
NS = 16
NW = NC * NS


def _sc_gather_chunk(table, tokens2d, c, n_chunks):
    V, D = table.shape
    Bt, L = tokens2d.shape
    seg = L // n_chunks
    Bc = Bt * seg
    tpw = Bc // NW
    CH = min(tpw, 128)
    n_ch = tpw // CH
    nbuf = min(n_ch, max(1, (500_000 - 4 * tpw) // (4 * CH * D)))
    mesh = plsc.VectorSubcoreMesh(core_axis_name="c", subcore_axis_name="s")

    @functools.partial(
        pl.kernel,
        mesh=mesh,
        out_type=jax.ShapeDtypeStruct((Bc, D), jnp.float32),
        scratch_types=(
            [pltpu.VMEM((tpw,), jnp.int32)]
            + [pltpu.VMEM((CH, D), jnp.float32)] * nbuf
            + [pltpu.SemaphoreType.DMA] * (2 * nbuf)
        ),
    )
    def gather_kernel(table_hbm, tok_hbm, out_hbm, idx_v, *bufs_sems):
        rows = bufs_sems[:nbuf]
        gsems = bufs_sems[nbuf:2 * nbuf]
        ssems = bufs_sems[2 * nbuf:3 * nbuf]
        wid = lax.axis_index("s") * NC + lax.axis_index("c")
        base = wid * tpw
        bi = base // seg
        col0 = c * seg + base % seg
        pltpu.sync_copy(tok_hbm.at[bi, pl.ds(col0, tpw)], idx_v)

        def start_gather(k):
            return pltpu.async_copy(
                table_hbm.at[idx_v.at[pl.ds(k * CH, CH)]], rows[k % nbuf],
                gsems[k % nbuf])

        gathers = [None] * nbuf
        stores = [None] * nbuf
        for k in range(min(nbuf, n_ch)):
            gathers[k] = start_gather(k)
        for k in range(n_ch):
            p = k % nbuf
            gathers[p].wait()
            stores[p] = pltpu.async_copy(
                rows[p], out_hbm.at[pl.ds(base + k * CH, CH)], ssems[p])
            if k + nbuf < n_ch:
                stores[p].wait()
                gathers[p] = start_gather(k + nbuf)
        for h in stores:
            if h is not None:
                h.wait()

    return gather_kernel(table, tokens2d)


def _tc_project_chunk(emb, Wb, b2, position, B, c, n_chunks, dest):
    Bc, D = emb.shape
    M = Wb.shape[0]
    L = position.shape[0]
    seg = L // n_chunks
    nb = Bc // seg

    def mm_kernel(*refs):
        emb_ref, w_ref, b_ref, pos_ref, out_ref = refs[-5:]
        acc = lax.dot_general(
            emb_ref[...].astype(jnp.bfloat16), w_ref[...],
            dimension_numbers=(((1,), (1,)), ((), ())),
            preferred_element_type=jnp.float32,
        )
        out_ref[...] = acc + b_ref[...] + pos_ref[...]

    data_specs = [
        pl.BlockSpec((seg, D), lambda k: (k, 0)),
        pl.BlockSpec((M, D), lambda k: (0, 0)),
        pl.BlockSpec((1, M), lambda k: (0, 0)),
        pl.BlockSpec((seg, M), lambda k: (c, 0)),
    ]
    if dest is None:
        in_specs, aliases, args = data_specs, {}, (emb, Wb, b2, position)
    else:
        in_specs = [pl.BlockSpec(memory_space=pl.ANY)] + data_specs
        aliases = {0: 0}
        args = (dest, emb, Wb, b2, position)

    return pl.pallas_call(
        mm_kernel,
        grid=(nb,),
        in_specs=in_specs,
        out_specs=pl.BlockSpec((seg, M), lambda k: (k * n_chunks + c, 0)),
        out_shape=jax.ShapeDtypeStruct((B, M), jnp.float32),
        input_output_aliases=aliases,
    )(*args)


def kernel(tokens, table, W, b, position):
    batch, seq = tokens.shape
    M = W.shape[0]
    B = batch * seq
    tokens2d = tokens.astype(jnp.int32)
    Wb = W.astype(jnp.bfloat16)
    b2 = b.reshape(1, M)

    n_chunks = 1
    embs = [_sc_gather_chunk(table, tokens2d, c, n_chunks)
            for c in range(n_chunks)]
    out = None
    for c in range(n_chunks):
        out = _tc_project_chunk(embs[c], Wb, b2, position, B, c, n_chunks, out)
    return out.reshape(batch, seq, M)
